# baseline (speedup 1.0000x reference)
import functools

import jax
import jax.numpy as jnp
from jax.experimental import pallas as pl

_N = 10000
_E = 160000
_NEG = 0.2


def _edge_mlp_body(z_ref, w_ref, b_ref, o_ref):
    z = z_ref[...]
    e = jnp.where(z > 0, z, jnp.exp(z) - 1.0)
    o_ref[...] = e @ w_ref[...] + b_ref[0]


def _edge_mlp(z, Cl2, c2b):
    BE = 3200
    grid = (z.shape[0] // BE,)
    out = pl.pallas_call(
        _edge_mlp_body,
        grid=grid,
        in_specs=[
            pl.BlockSpec((BE, z.shape[1]), lambda i: (i, 0)),
            pl.BlockSpec((z.shape[1], 1), lambda i: (0, 0)),
            pl.BlockSpec((1,), lambda i: (0,)),
        ],
        out_specs=pl.BlockSpec((BE, 1), lambda i: (i, 0)),
        out_shape=jax.ShapeDtypeStruct((z.shape[0], 1), jnp.float32),
    )(z, Cl2, c2b)
    return out[:, 0]


def _gat(x, src, dst, edge_attr, Wl, bl, Wr, br, We, att, H, C):
    n = x.shape[0]
    xl = (x @ Wl + bl).reshape(n, H, C)
    xr = (x @ Wr + br).reshape(n, H, C)
    e = xl[src] + xr[dst] + (edge_attr @ We).reshape(-1, H, C)
    e = jnp.where(e > 0, e, _NEG * e)
    alpha = (e * att[None, :, :]).sum(-1)
    amax = jax.ops.segment_max(alpha, dst, num_segments=n)
    amax = jnp.where(jnp.isfinite(amax), amax, 0.0)
    ex = jnp.exp(alpha - amax[dst])
    den = jax.ops.segment_sum(ex, dst, num_segments=n)
    a = ex / (den[dst] + 1e-16)
    out = jax.ops.segment_sum(xl[src] * a[:, :, None], dst, num_segments=n)
    return out


def kernel(x, edge_index, edge_attr, Wl1, bl1, Wr1, br1, We1, att1, bias1,
           Wl2, bl2, Wr2, br2, We2, att2, bias2,
           A1, a1b, A2, a2b, Cl1, c1b, Cl2, c2b):
    src = edge_index[0]
    dst = edge_index[1]
    h = _gat(x, src, dst, edge_attr, Wl1, bl1, Wr1, br1, We1, att1, 4, 64)
    h = h.reshape(_N, 256) + bias1
    h = jax.nn.elu(h)
    h2 = _gat(h, src, dst, edge_attr, Wl2, bl2, Wr2, br2, We2, att2, 1, 256)
    h2 = h2.reshape(_N, 256) + bias2
    C2 = 256
    Ps = h2 @ Cl1[:C2]
    Pd = h2 @ Cl1[C2:2 * C2]
    Pe = edge_attr @ Cl1[2 * C2:] + c1b
    z = Ps[src] + Pd[dst] + Pe
    return _edge_mlp(z, Cl2, c2b)


# speedup vs baseline: 1.3273x; 1.3273x over previous
import functools

import jax
import jax.numpy as jnp
from jax import lax
from jax.experimental import pallas as pl
from jax.experimental.pallas import tpu as pltpu
from jax.experimental.pallas import tpu_sc as plsc

_N = 10000
_E = 160000
_D = 256
_NEG = 0.2

_CH = 64
_NCHUNK = _E // _CH
_NW = 32
_HALF = 5008
_ROWS_PER_TILE = _HALF // 16


def _mm_body(x_ref, w_ref, b_ref, o_ref):
    o_ref[...] = (
        jnp.dot(x_ref[...], w_ref[...], preferred_element_type=jnp.float32)
        + b_ref[...]
    )


def _mm(x, w, b, bm):
    m, k = x.shape
    n = w.shape[1]
    return pl.pallas_call(
        _mm_body,
        grid=(m // bm,),
        in_specs=[
            pl.BlockSpec((bm, k), lambda i: (i, 0)),
            pl.BlockSpec((k, n), lambda i: (0, 0)),
            pl.BlockSpec((1, n), lambda i: (0, 0)),
        ],
        out_specs=pl.BlockSpec((bm, n), lambda i: (i, 0)),
        out_shape=jax.ShapeDtypeStruct((m, n), jnp.float32),
    )(x, w, b.reshape(1, n))


def _make_scores(heads, elu):
    vph = 16 // heads
    mesh = plsc.VectorSubcoreMesh(core_axis_name="c", subcore_axis_name="s")

    @functools.partial(
        pl.kernel,
        mesh=mesh,
        compiler_params=pltpu.CompilerParams(use_tc_tiling_on_sc=False, needs_layout_passes=False),
        out_type=jax.ShapeDtypeStruct((heads, _E), jnp.float32),
        scratch_types=[
            pltpu.VMEM((_CH,), jnp.int32),
            pltpu.VMEM((_CH,), jnp.int32),
            pltpu.VMEM((_CH, _D), jnp.float32),
            pltpu.VMEM((_CH, _D), jnp.float32),
            pltpu.VMEM((_CH, _D), jnp.float32),
            pltpu.VMEM((heads, _CH), jnp.float32),
            pltpu.VMEM((_D,), jnp.float32),
            pltpu.SemaphoreType.DMA,
        ],
    )
    def scores(xl_hbm, xr_hbm, we_hbm, src_hbm, dst_hbm, attw_hbm, alpha_hbm,
               idx_s, idx_d, xl_v, xr_v, we_v, alpha_v, attw_v, sem):
        c = lax.axis_index("c")
        s = lax.axis_index("s")
        wid = s * 2 + c
        pltpu.sync_copy(attw_hbm, attw_v)
        nch = 78 + (wid < (_NCHUNK - 78 * _NW)).astype(jnp.int32)

        def chunk_body(i, carry):
            base = (wid + _NW * i) * _CH
            pltpu.sync_copy(src_hbm.at[pl.ds(base, _CH)], idx_s)
            pltpu.sync_copy(dst_hbm.at[pl.ds(base, _CH)], idx_d)
            pltpu.async_copy(xl_hbm.at[idx_s], xl_v, sem).wait()
            pltpu.async_copy(xr_hbm.at[idx_d], xr_v, sem).wait()
            pltpu.sync_copy(we_hbm.at[pl.ds(base, _CH)], we_v)

            for g in range(_CH // 16):
                eids = lax.iota(jnp.int32, 16) + g * 16
                for h in range(heads):
                    cw = vph * 16

                    def feat_body(f, acc, eids=eids):
                        fsplat = jnp.zeros((16,), jnp.int32) + f
                        t = (plsc.load_gather(xl_v, [eids, fsplat])
                             + plsc.load_gather(xr_v, [eids, fsplat])
                             + plsc.load_gather(we_v, [eids, fsplat]))
                        if elu:
                            t = jnp.where(t > 0, t, jnp.exp(t) - 1.0)
                        else:
                            t = jnp.where(t > 0, t, _NEG * t)
                        aw = plsc.load_gather(attw_v, [fsplat])
                        return acc + t * aw

                    acc = lax.fori_loop(h * cw, (h + 1) * cw, feat_body,
                                        jnp.zeros((16,), jnp.float32))
                    alpha_v[h, pl.ds(g * 16, 16)] = acc
            for h in range(heads):
                pltpu.sync_copy(alpha_v.at[h], alpha_hbm.at[h, pl.ds(base, _CH)])
            return carry

        lax.fori_loop(0, nch, chunk_body, 0)

    return scores


_scores_h4 = _make_scores(4, elu=False)
_scores_h1 = _make_scores(1, elu=False)
_scores_cls = _make_scores(1, elu=True)


def _make_agg(heads):
    vph = 16 // heads
    mesh = plsc.VectorSubcoreMesh(core_axis_name="c", subcore_axis_name="s")

    @functools.partial(
        pl.kernel,
        mesh=mesh,
        compiler_params=pltpu.CompilerParams(use_tc_tiling_on_sc=False, needs_layout_passes=False),
        out_type=jax.ShapeDtypeStruct((_N, _D), jnp.float32),
        scratch_types=[
            pltpu.VMEM((_CH,), jnp.int32),
            pltpu.VMEM((_CH,), jnp.int32),
            pltpu.VMEM((_CH, _D), jnp.float32),
            pltpu.VMEM((heads, _CH), jnp.float32),
            pltpu.VMEM((_CH, _D), jnp.float32),
            pltpu.VMEM_SHARED((_HALF + 16, _D), jnp.float32),
            pltpu.SemaphoreType.DMA,
        ],
    )
    def agg(xl_hbm, src_hbm, dst_hbm, at_hbm, out_hbm,
            idx_s, idx_d, rows_v, a_v, zbuf, acc, sem):
        c = lax.axis_index("c")
        s = lax.axis_index("s")

        def zrow(r, carry):
            for v in range(16):
                zbuf[r, pl.ds(v * 16, 16)] = jnp.zeros((16,), jnp.float32)
            return carry

        lax.fori_loop(0, _CH, zrow, 0)
        zoff = s * _ROWS_PER_TILE
        for off in range(0, _ROWS_PER_TILE, _CH):
            sz = min(_CH, _ROWS_PER_TILE - off)
            pltpu.sync_copy(zbuf.at[pl.ds(0, sz)], acc.at[pl.ds(zoff + off, sz)])
        plsc.subcore_barrier()

        nch = 156 + (s < (_NCHUNK - 156 * 16)).astype(jnp.int32)

        def chunk_body(i, carry):
            base = (s + 16 * i) * _CH
            pltpu.sync_copy(src_hbm.at[pl.ds(base, _CH)], idx_s)
            pltpu.sync_copy(dst_hbm.at[pl.ds(base, _CH)], idx_d)
            pltpu.async_copy(xl_hbm.at[idx_s], rows_v, sem).wait()
            for h in range(heads):
                pltpu.sync_copy(at_hbm.at[h, pl.ds(base, _CH)], a_v.at[h])

            for g in range(_CH // 16):
                eids = lax.iota(jnp.int32, 16) + g * 16
                for h in range(heads):
                    avec = a_v[h, pl.ds(g * 16, 16)]
                    cw = vph * 16

                    def feat_body(f, carry2, eids=eids, avec=avec):
                        fsplat = jnp.zeros((16,), jnp.int32) + f
                        vec = plsc.load_gather(rows_v, [eids, fsplat])
                        plsc.store_scatter(rows_v, [eids, fsplat], vec * avec)
                        return carry2

                    lax.fori_loop(h * cw, (h + 1) * cw, feat_body, 0)

            for v in range(_CH // 16):
                sl = pl.ds(v * 16, 16)
                loc = idx_d[sl] - c * _HALF
                ok = (loc >= 0) & (loc < _HALF)
                idx_d[sl] = jnp.where(ok, loc, _HALF)
            pltpu.sync_copy(rows_v, acc.at[idx_d], add=True)
            return carry

        lax.fori_loop(0, nch, chunk_body, 0)
        plsc.subcore_barrier()

        last = jnp.logical_and(c == 1, s == 15)
        gbase = c * _HALF + s * _ROWS_PER_TILE

        @pl.when(jnp.logical_not(last))
        def _():
            pltpu.sync_copy(
                acc.at[pl.ds(s * _ROWS_PER_TILE, _ROWS_PER_TILE)],
                out_hbm.at[pl.ds(gbase, _ROWS_PER_TILE)],
            )

        @pl.when(last)
        def _():
            tail = _N - (_HALF + 15 * _ROWS_PER_TILE)
            pltpu.sync_copy(
                acc.at[pl.ds(15 * _ROWS_PER_TILE, tail)],
                out_hbm.at[pl.ds(_HALF + 15 * _ROWS_PER_TILE, tail)],
            )

    return agg


_agg_h4 = _make_agg(4)
_agg_h1 = _make_agg(1)


def _norm(alpha_t, dst):
    alpha = alpha_t.T
    amax = jax.ops.segment_max(alpha, dst, num_segments=_N)
    amax = jnp.where(jnp.isfinite(amax), amax, 0.0)
    ex = jnp.exp(alpha - amax[dst])
    den = jax.ops.segment_sum(ex, dst, num_segments=_N)
    return (ex / (den[dst] + 1e-16)).T


def kernel(x, edge_index, edge_attr, Wl1, bl1, Wr1, br1, We1, att1, bias1,
           Wl2, bl2, Wr2, br2, We2, att2, bias2,
           A1, a1b, A2, a2b, Cl1, c1b, Cl2, c2b):
    src = edge_index[0]
    dst = edge_index[1]
    zb = jnp.zeros((_D,), jnp.float32)

    xl1 = _mm(x, Wl1, bl1, 400)
    xr1 = _mm(x, Wr1, br1, 400)
    we1 = _mm(edge_attr, We1, zb, 2000)
    alpha1 = _scores_h4(xl1, xr1, we1, src, dst, att1.reshape(-1))
    a1 = _norm(alpha1, dst)
    h = _agg_h4(xl1, src, dst, a1)
    h = jax.nn.elu(h + bias1)

    xl2 = _mm(h, Wl2, bl2, 400)
    xr2 = _mm(h, Wr2, br2, 400)
    we2 = _mm(edge_attr, We2, zb, 2000)
    alpha2 = _scores_h1(xl2, xr2, we2, src, dst, att2.reshape(-1))
    a2 = _norm(alpha2, dst)
    h2 = _agg_h1(xl2, src, dst, a2)
    h2 = h2 + bias2

    ps = _mm(h2, Cl1[:_D], zb, 400)
    pd = _mm(h2, Cl1[_D:2 * _D], zb, 400)
    pe = _mm(edge_attr, Cl1[2 * _D:], c1b, 2000)
    out = _scores_cls(ps, pd, pe, src, dst, Cl2[:, 0])
    return out[0] + c2b[0]
